# Initial kernel scaffold; baseline (speedup 1.0000x reference)
import jax
import jax.numpy as jnp
from jax.experimental import pallas as pl


def kernel(x, edge_index, edge_attr, batch, params):
    raise NotImplementedError("write your pallas kernel here")


# speedup vs baseline: 1.9155x; 1.9155x over previous
import functools

import jax
import jax.numpy as jnp
from jax import lax
from jax.experimental import pallas as pl
from jax.experimental.pallas import tpu as pltpu
from jax.experimental.pallas import tpu_sc as plsc

N = 10000
E = 160000
D = 256
H = 256
ED = 16
G = 64
HALF = 128

NTILES = 16
EPT = E // NTILES
EC = 80
NCH = EPT // EC
RC = 80
NWB = N // RC
WBPT = -(-NWB // NTILES)


def _sc_body(l, src_hbm, dst_hbm, x0_hbm, x1_hbm, e_hbm, out0_hbm, out1_hbm,
             idx_s, idx_d, xrows, erows, wb, aggr_sh, sem):
    c = lax.axis_index("c")
    tid = lax.axis_index("s")

    zeros16 = jnp.zeros((16,), jnp.float32)

    def _zero_row(r, _):
        for k in range(HALF // 16):
            wb[r, pl.ds(k * 16, 16)] = zeros16
        return 0

    lax.fori_loop(0, RC, _zero_row, 0)

    def _zero_cp(k, _):
        j = k * NTILES + tid

        @pl.when(j < NWB)
        def _():
            pltpu.sync_copy(wb, aggr_sh.at[pl.ds(j * RC, RC)])

        return 0

    lax.fori_loop(0, WBPT, _zero_cp, 0)
    plsc.subcore_barrier()

    ebase = tid * EPT
    e_row0 = 2 * l * E

    def _chunk(i, _):
        off = ebase + i * EC
        pltpu.sync_copy(src_hbm.at[pl.ds(off, EC)], idx_s)
        pltpu.sync_copy(dst_hbm.at[pl.ds(off, EC)], idx_d)

        @pl.when(c == 0)
        def _():
            pltpu.async_copy(x0_hbm.at[idx_s], xrows, sem).wait()
            pltpu.sync_copy(e_hbm.at[pl.ds(e_row0 + off, EC)], erows)

        @pl.when(c == 1)
        def _():
            pltpu.async_copy(x1_hbm.at[idx_s], xrows, sem).wait()
            pltpu.sync_copy(e_hbm.at[pl.ds(e_row0 + E + off, EC)], erows)

        def _comp(r, _):
            for k in range(HALF // 16):
                sl = pl.ds(k * 16, 16)
                xrows[r, sl] = jnp.maximum(xrows[r, sl] + erows[r, sl], 0.0)
            return 0

        lax.fori_loop(0, EC, _comp, 0)
        pltpu.sync_copy(xrows, aggr_sh.at[idx_d], add=True)
        return 0

    lax.fori_loop(0, NCH, _chunk, 0)
    plsc.subcore_barrier()

    def _wb(k, _):
        j = k * NTILES + tid

        @pl.when(j < NWB)
        def _():
            r0 = j * RC
            pltpu.sync_copy(aggr_sh.at[pl.ds(r0, RC)], wb)

            @pl.when(c == 0)
            def _():
                pltpu.sync_copy(wb, out0_hbm.at[pl.ds(r0, RC)])

            @pl.when(c == 1)
            def _():
                pltpu.sync_copy(wb, out1_hbm.at[pl.ds(r0, RC)])

        return 0

    lax.fori_loop(0, WBPT, _wb, 0)


def _make_sc_aggregate(l):
    mesh = plsc.VectorSubcoreMesh(
        core_axis_name="c", subcore_axis_name="s", num_cores=2,
        num_subcores=NTILES,
    )
    return pl.kernel(
        functools.partial(_sc_body, l),
        out_type=(
            jax.ShapeDtypeStruct((N, HALF), jnp.float32),
            jax.ShapeDtypeStruct((N, HALF), jnp.float32),
        ),
        mesh=mesh,
        scratch_types=[
            pltpu.VMEM((EC,), jnp.int32),
            pltpu.VMEM((EC,), jnp.int32),
            pltpu.VMEM((EC, HALF), jnp.float32),
            pltpu.VMEM((EC, HALF), jnp.float32),
            pltpu.VMEM((RC, HALF), jnp.float32),
            pltpu.VMEM_SHARED((N, HALF), jnp.float32),
            pltpu.SemaphoreType.DMA,
        ],
    )


EB = 8000


def _edge_mm_body(a_ref, w_ref, b_ref, o_ref):
    o_ref[0] = (
        jnp.dot(a_ref[...], w_ref[...], preferred_element_type=jnp.float32)
        + b_ref[...]
    )


def _edge_features(edge_attr, wcat, bcat):
    return pl.pallas_call(
        _edge_mm_body,
        grid=(E // EB, 10),
        in_specs=[
            pl.BlockSpec((EB, ED), lambda i, j: (i, 0)),
            pl.BlockSpec((ED, HALF), lambda i, j: (0, j)),
            pl.BlockSpec((1, HALF), lambda i, j: (0, j)),
        ],
        out_specs=pl.BlockSpec((1, EB, HALF), lambda i, j: (j, i, 0)),
        out_shape=jax.ShapeDtypeStruct((10, E, HALF), jnp.float32),
    )(edge_attr, wcat, bcat)


NB = 400


def _mlp_body(x0, x1, a0, a1, w1, b1, gs, be, w2, b2, o0, o1):
    t = jnp.concatenate([x0[...] + a0[...], x1[...] + a1[...]], axis=1)
    z = jnp.dot(t, w1[...], preferred_element_type=jnp.float32) + b1[...]
    z = jnp.maximum(z * gs[...] + be[...], 0.0)
    o = jnp.maximum(
        jnp.dot(z, w2[...], preferred_element_type=jnp.float32) + b2[...], 0.0
    )
    o0[...] = o[:, :HALF]
    o1[...] = o[:, HALF:]


def _node_mlp(x0, x1, a0, a1, w1, b1, gs, be, w2, b2):
    row = pl.BlockSpec((NB, HALF), lambda i: (i, 0))
    full = lambda shape: pl.BlockSpec(shape, lambda i: (0, 0))
    return pl.pallas_call(
        _mlp_body,
        grid=(N // NB,),
        in_specs=[
            row, row, row, row,
            full((H, H)), full((1, H)), full((1, H)), full((1, H)),
            full((H, H)), full((1, H)),
        ],
        out_specs=[row, row],
        out_shape=(
            jax.ShapeDtypeStruct((N, HALF), jnp.float32),
            jax.ShapeDtypeStruct((N, HALF), jnp.float32),
        ),
    )(x0, x1, a0, a1, w1, b1, gs, be, w2, b2)


PB = 400
NPB = N // PB


def _pool_body(b_ref, *rest):
    hs = rest[:10]
    l1w, l1b, l2w, l2b, o_ref, acc, cnt = rest[10:]
    i = pl.program_id(0)

    @pl.when(i == 0)
    def _():
        acc[...] = jnp.zeros((G, 5 * H), jnp.float32)
        cnt[...] = jnp.zeros((G, HALF), jnp.float32)
        o_ref[...] = jnp.zeros((G, HALF), jnp.float32)

    bvec = b_ref[0, 0, :].reshape(PB, 1)
    onehot = (bvec == lax.broadcasted_iota(jnp.int32, (PB, G), 1)).astype(
        jnp.float32
    )
    hcat = jnp.concatenate([h[...] for h in hs], axis=1)
    acc[...] += lax.dot_general(
        onehot, hcat, (((0,), (0,)), ((), ())),
        preferred_element_type=jnp.float32,
    )
    cnt[...] += lax.dot_general(
        onehot, jnp.ones((PB, HALF), jnp.float32), (((0,), (0,)), ((), ())),
        preferred_element_type=jnp.float32,
    )

    @pl.when(i == NPB - 1)
    def _():
        inv = 1.0 / jnp.maximum(cnt[:, 0:1], 1.0)
        m = acc[...] * inv
        z = jnp.maximum(
            jnp.dot(m, l1w[...], preferred_element_type=jnp.float32) + l1b[...],
            0.0,
        )
        o_ref[...] = (
            jnp.dot(z, l2w[...], preferred_element_type=jnp.float32) + l2b[...]
        )


def _pool_head(batch_r, hs, l1w, l1b, l2w_pad, l2b_pad):
    row = pl.BlockSpec((PB, HALF), lambda i: (i, 0))
    full = lambda shape: pl.BlockSpec(shape, lambda i: (0, 0))
    return pl.pallas_call(
        _pool_body,
        grid=(NPB,),
        in_specs=[pl.BlockSpec((1, 1, PB), lambda i: (i, 0, 0))]
        + [row] * 10
        + [full((5 * H, 5 * H)), full((1, 5 * H)), full((5 * H, HALF)),
           full((1, HALF))],
        out_specs=full((G, HALF)),
        out_shape=jax.ShapeDtypeStruct((G, HALF), jnp.float32),
        scratch_shapes=[
            pltpu.VMEM((G, 5 * H), jnp.float32),
            pltpu.VMEM((G, HALF), jnp.float32),
        ],
    )(batch_r, *hs, l1w, l1b, l2w_pad, l2b_pad)


def kernel(x, edge_index, edge_attr, batch, params):
    src = edge_index[0].astype(jnp.int32)
    dst = edge_index[1].astype(jnp.int32)
    batch_r = batch.astype(jnp.int32).reshape(NPB, 1, PB)

    layers = [params[f'c{k}'] for k in range(1, 6)]
    wcat = jnp.concatenate([p['elw'] for p in layers], axis=1)
    bcat = jnp.concatenate([p['elb'] for p in layers]).reshape(1, 5 * H)

    e_all = _edge_features(edge_attr, wcat, bcat)
    e_flat = e_all.reshape(10 * E, HALF)

    h0 = x[:, :HALF]
    h1 = x[:, HALF:]
    bn = 1.0 / jnp.sqrt(jnp.float32(1.0 + 1e-5))

    hs = []
    for l, p in enumerate(layers):
        a0, a1 = _make_sc_aggregate(l)(src, dst, h0, h1, e_flat)
        h0, h1 = _node_mlp(
            h0, h1, a0, a1,
            p['w1'], p['b1'].reshape(1, H),
            (p['g'] * bn).reshape(1, H), p['be'].reshape(1, H),
            p['w2'], p['b2'].reshape(1, H),
        )
        hs.extend([h0, h1])

    l2w_pad = jnp.pad(params['l2w'], ((0, 0), (0, HALF - 1)))
    l2b_pad = jnp.pad(params['l2b'], (0, HALF - 1)).reshape(1, HALF)
    out = _pool_head(
        batch_r, hs, params['l1w'], params['l1b'].reshape(1, 5 * H),
        l2w_pad, l2b_pad,
    )
    return out[:, 0]


# speedup vs baseline: 3.8345x; 2.0018x over previous
import functools

import jax
import jax.numpy as jnp
from jax import lax
from jax.experimental import pallas as pl
from jax.experimental.pallas import tpu as pltpu
from jax.experimental.pallas import tpu_sc as plsc

N = 10000
E = 160000
D = 256
H = 256
ED = 16
G = 64
HALF = 128

NTILES = 16
EPT = E // NTILES
EC = 40
NCH = EPT // EC
W = 50
NG = NCH // W
RC = 40
NWB = N // RC
WBPT = -(-NWB // NTILES)


def _sc_body(l, src_hbm, dst_hbm, x0_hbm, x1_hbm, e_hbm, out0_hbm, out1_hbm,
             srw, dsw, xr0, xr1, er0, er1, mg0, mg1, wb, aggr_sh,
             is0, gs0, gs1, es0, es1, ss0, ss1):
    c = lax.axis_index("c")
    tid = lax.axis_index("s")

    zeros16 = jnp.zeros((16,), jnp.float32)

    def _zero_row(r, _):
        for k in range(HALF // 16):
            wb[r, pl.ds(k * 16, 16)] = zeros16
        return 0

    lax.fori_loop(0, RC, _zero_row, 0)

    def _zero_cp(k, _):
        j = k * NTILES + tid

        @pl.when(j < NWB)
        def _():
            pltpu.sync_copy(wb, aggr_sh.at[pl.ds(j * RC, RC)])

        return 0

    lax.fori_loop(0, WBPT, _zero_cp, 0)
    plsc.subcore_barrier()

    e_base = (2 * l + c) * E + tid * EPT
    bufs = ((xr0, er0, mg0, gs0, es0, ss0), (xr1, er1, mg1, gs1, es1, ss1))

    def _issue_loads(j, i, buf):
        xr, er, _, gs, es, _ = buf

        @pl.when(c == 0)
        def _():
            pltpu.async_copy(x0_hbm.at[srw.at[j]], xr, gs)

        @pl.when(c == 1)
        def _():
            pltpu.async_copy(x1_hbm.at[srw.at[j]], xr, gs)

        pltpu.async_copy(e_hbm.at[pl.ds(e_base + i * EC, EC)], er, es)

    def _group(g, _):
        g0 = g * W
        pltpu.async_copy(src_hbm.at[tid, g], srw, is0)
        pltpu.async_copy(dst_hbm.at[tid, g], dsw, is0)
        pltpu.make_async_copy(src_hbm.at[tid, 0], srw, is0).wait()
        pltpu.make_async_copy(dst_hbm.at[tid, 0], dsw, is0).wait()
        _issue_loads(0, g0 + 0, bufs[0])
        _issue_loads(1, g0 + 1, bufs[1])

        def _pair(k, _):
            for b in range(2):
                xr, er, mg, gs, es, ss = bufs[b]
                j = 2 * k + b
                i = g0 + j
                pltpu.make_async_copy(x0_hbm.at[srw.at[0]], xr, gs).wait()
                pltpu.make_async_copy(e_hbm.at[pl.ds(0, EC)], er, es).wait()

                @pl.when(k >= 1)
                def _():
                    pltpu.make_async_copy(
                        mg, aggr_sh.at[dsw.at[0]], ss).wait()

                @plsc.parallel_loop(0, EC, unroll=2)
                def _(r):
                    for kk in range(HALF // 16):
                        sl = pl.ds(kk * 16, 16)
                        mg[r, sl] = jnp.maximum(xr[r, sl] + er[r, sl], 0.0)

                @pl.when(k < W // 2 - 1)
                def _():
                    _issue_loads(j + 2, i + 2, bufs[b])

                pltpu.async_copy(mg, aggr_sh.at[dsw.at[j]], ss, add=True)
            return 0

        lax.fori_loop(0, W // 2, _pair, 0)
        pltpu.make_async_copy(mg0, aggr_sh.at[dsw.at[0]], ss0).wait()
        pltpu.make_async_copy(mg1, aggr_sh.at[dsw.at[0]], ss1).wait()
        return 0

    lax.fori_loop(0, NG, _group, 0)
    plsc.subcore_barrier()

    def _wb(k, _):
        j = k * NTILES + tid

        @pl.when(j < NWB)
        def _():
            r0 = j * RC
            pltpu.sync_copy(aggr_sh.at[pl.ds(r0, RC)], wb)

            @pl.when(c == 0)
            def _():
                pltpu.sync_copy(wb, out0_hbm.at[pl.ds(r0, RC)])

            @pl.when(c == 1)
            def _():
                pltpu.sync_copy(wb, out1_hbm.at[pl.ds(r0, RC)])

        return 0

    lax.fori_loop(0, WBPT, _wb, 0)


def _make_sc_aggregate(l):
    mesh = plsc.VectorSubcoreMesh(
        core_axis_name="c", subcore_axis_name="s", num_cores=2,
        num_subcores=NTILES,
    )
    return pl.kernel(
        functools.partial(_sc_body, l),
        out_type=(
            jax.ShapeDtypeStruct((N, HALF), jnp.float32),
            jax.ShapeDtypeStruct((N, HALF), jnp.float32),
        ),
        mesh=mesh,
        scratch_types=[
            pltpu.VMEM((W, EC), jnp.int32),
            pltpu.VMEM((W, EC), jnp.int32),
            pltpu.VMEM((EC, HALF), jnp.float32),
            pltpu.VMEM((EC, HALF), jnp.float32),
            pltpu.VMEM((EC, HALF), jnp.float32),
            pltpu.VMEM((EC, HALF), jnp.float32),
            pltpu.VMEM((EC, HALF), jnp.float32),
            pltpu.VMEM((EC, HALF), jnp.float32),
            pltpu.VMEM((RC, HALF), jnp.float32),
            pltpu.VMEM_SHARED((N, HALF), jnp.float32),
            pltpu.SemaphoreType.DMA,
            pltpu.SemaphoreType.DMA,
            pltpu.SemaphoreType.DMA,
            pltpu.SemaphoreType.DMA,
            pltpu.SemaphoreType.DMA,
            pltpu.SemaphoreType.DMA,
            pltpu.SemaphoreType.DMA,
        ],
    )


EB = 8000


def _edge_mm_body(a_ref, w_ref, b_ref, o_ref):
    o_ref[0] = (
        jnp.dot(a_ref[...], w_ref[...], preferred_element_type=jnp.float32)
        + b_ref[...]
    )


def _edge_features(edge_attr, wcat, bcat):
    return pl.pallas_call(
        _edge_mm_body,
        grid=(E // EB, 10),
        in_specs=[
            pl.BlockSpec((EB, ED), lambda i, j: (i, 0)),
            pl.BlockSpec((ED, HALF), lambda i, j: (0, j)),
            pl.BlockSpec((1, HALF), lambda i, j: (0, j)),
        ],
        out_specs=pl.BlockSpec((1, EB, HALF), lambda i, j: (j, i, 0)),
        out_shape=jax.ShapeDtypeStruct((10, E, HALF), jnp.float32),
    )(edge_attr, wcat, bcat)


NB = 400


def _mlp_body(x0, x1, a0, a1, w1, b1, gs, be, w2, b2, o0, o1):
    t = jnp.concatenate([x0[...] + a0[...], x1[...] + a1[...]], axis=1)
    z = jnp.dot(t, w1[...], preferred_element_type=jnp.float32) + b1[...]
    z = jnp.maximum(z * gs[...] + be[...], 0.0)
    o = jnp.maximum(
        jnp.dot(z, w2[...], preferred_element_type=jnp.float32) + b2[...], 0.0
    )
    o0[...] = o[:, :HALF]
    o1[...] = o[:, HALF:]


def _node_mlp(x0, x1, a0, a1, w1, b1, gs, be, w2, b2):
    row = pl.BlockSpec((NB, HALF), lambda i: (i, 0))
    full = lambda shape: pl.BlockSpec(shape, lambda i: (0, 0))
    return pl.pallas_call(
        _mlp_body,
        grid=(N // NB,),
        in_specs=[
            row, row, row, row,
            full((H, H)), full((1, H)), full((1, H)), full((1, H)),
            full((H, H)), full((1, H)),
        ],
        out_specs=[row, row],
        out_shape=(
            jax.ShapeDtypeStruct((N, HALF), jnp.float32),
            jax.ShapeDtypeStruct((N, HALF), jnp.float32),
        ),
    )(x0, x1, a0, a1, w1, b1, gs, be, w2, b2)


PB = 400
NPB = N // PB


def _pool_body(b_ref, *rest):
    hs = rest[:10]
    l1w, l1b, l2w, l2b, o_ref, acc, cnt = rest[10:]
    i = pl.program_id(0)

    @pl.when(i == 0)
    def _():
        acc[...] = jnp.zeros((G, 5 * H), jnp.float32)
        cnt[...] = jnp.zeros((G, HALF), jnp.float32)
        o_ref[...] = jnp.zeros((G, HALF), jnp.float32)

    bvec = b_ref[0, 0, :].reshape(PB, 1)
    onehot = (bvec == lax.broadcasted_iota(jnp.int32, (PB, G), 1)).astype(
        jnp.float32
    )
    hcat = jnp.concatenate([h[...] for h in hs], axis=1)
    acc[...] += lax.dot_general(
        onehot, hcat, (((0,), (0,)), ((), ())),
        preferred_element_type=jnp.float32,
    )
    cnt[...] += lax.dot_general(
        onehot, jnp.ones((PB, HALF), jnp.float32), (((0,), (0,)), ((), ())),
        preferred_element_type=jnp.float32,
    )

    @pl.when(i == NPB - 1)
    def _():
        inv = 1.0 / jnp.maximum(cnt[:, 0:1], 1.0)
        m = acc[...] * inv
        z = jnp.maximum(
            jnp.dot(m, l1w[...], preferred_element_type=jnp.float32) + l1b[...],
            0.0,
        )
        o_ref[...] = (
            jnp.dot(z, l2w[...], preferred_element_type=jnp.float32) + l2b[...]
        )


def _pool_head(batch_r, hs, l1w, l1b, l2w_pad, l2b_pad):
    row = pl.BlockSpec((PB, HALF), lambda i: (i, 0))
    full = lambda shape: pl.BlockSpec(shape, lambda i: (0, 0))
    return pl.pallas_call(
        _pool_body,
        grid=(NPB,),
        in_specs=[pl.BlockSpec((1, 1, PB), lambda i: (i, 0, 0))]
        + [row] * 10
        + [full((5 * H, 5 * H)), full((1, 5 * H)), full((5 * H, HALF)),
           full((1, HALF))],
        out_specs=full((G, HALF)),
        out_shape=jax.ShapeDtypeStruct((G, HALF), jnp.float32),
        scratch_shapes=[
            pltpu.VMEM((G, 5 * H), jnp.float32),
            pltpu.VMEM((G, HALF), jnp.float32),
        ],
    )(batch_r, *hs, l1w, l1b, l2w_pad, l2b_pad)


def kernel(x, edge_index, edge_attr, batch, params):
    src = edge_index[0].astype(jnp.int32).reshape(NTILES, NG, W, EC)
    dst = edge_index[1].astype(jnp.int32).reshape(NTILES, NG, W, EC)
    batch_r = batch.astype(jnp.int32).reshape(NPB, 1, PB)

    layers = [params[f'c{k}'] for k in range(1, 6)]
    wcat = jnp.concatenate([p['elw'] for p in layers], axis=1)
    bcat = jnp.concatenate([p['elb'] for p in layers]).reshape(1, 5 * H)

    e_all = _edge_features(edge_attr, wcat, bcat)
    e_flat = e_all.reshape(10 * E, HALF)

    h0 = x[:, :HALF]
    h1 = x[:, HALF:]
    bn = 1.0 / jnp.sqrt(jnp.float32(1.0 + 1e-5))

    hs = []
    for l, p in enumerate(layers):
        a0, a1 = _make_sc_aggregate(l)(src, dst, h0, h1, e_flat)
        h0, h1 = _node_mlp(
            h0, h1, a0, a1,
            p['w1'], p['b1'].reshape(1, H),
            (p['g'] * bn).reshape(1, H), p['be'].reshape(1, H),
            p['w2'], p['b2'].reshape(1, H),
        )
        hs.extend([h0, h1])

    l2w_pad = jnp.pad(params['l2w'], ((0, 0), (0, HALF - 1)))
    l2b_pad = jnp.pad(params['l2b'], (0, HALF - 1)).reshape(1, HALF)
    out = _pool_head(
        batch_r, hs, params['l1w'], params['l1b'].reshape(1, 5 * H),
        l2w_pad, l2b_pad,
    )
    return out[:, 0]
